# baseline (speedup 1.0000x reference)
import functools

import jax
import jax.numpy as jnp
from jax import lax
from jax.experimental import pallas as pl
from jax.experimental.pallas import tpu as pltpu
from jax.experimental.pallas import tpu_sc as plsc

NC = 2
NS = 16
NW = NC * NS

B = 4
T = 2048
D = 768
LANES = 16
DV = D // LANES

TW = T // NW
CH = 32
PERB = TW // CH
NCHUNK = B * PERB
NBUF = 3


def _emb_body(
    idx_hbm, pos_hbm, tok_hbm, out_hbm,
    idx_v, pos_v, rows_v,
    isem, gsem0, gsem1, wsem0, wsem1, psem,
):
    wid = lax.axis_index("s") * NC + lax.axis_index("c")
    t0 = wid * TW
    gsems = (gsem0, gsem1)
    wsems = (wsem0, wsem1)

    def row0(h):
        return (h // PERB) * T + t0 + (h % PERB) * CH

    i0 = pltpu.async_copy(
        idx_hbm.at[pl.ds(row0(0), CH)], idx_v.at[pl.ds(0, CH)], isem
    )

    seq = [b * PERB + q for q in range(PERB) for b in range(B)]
    bufno = {h: i % NBUF for i, h in enumerate(seq)}
    semno = {h: i % 2 for i, h in enumerate(seq)}

    def gathers(h):
        return [
            pltpu.async_copy(
                tok_hbm.at[idx_v.at[pl.ds(h * CH, CH)]],
                rows_v.at[bufno[h]],
                gsems[semno[h]],
            )
        ]

    i0.wait()
    g = {seq[0]: gathers(seq[0])}
    pq = {0: pltpu.async_copy(pos_hbm.at[pl.ds(t0, CH)], pos_v.at[0], psem)}
    irest = [
        pltpu.async_copy(
            idx_hbm.at[pl.ds(b * T + t0 + (CH if b == 0 else 0), TW - (CH if b == 0 else 0))],
            idx_v.at[pl.ds(b * TW + (CH if b == 0 else 0), TW - (CH if b == 0 else 0))],
            isem,
        )
        for b in range(B)
    ]
    for cp in irest:
        cp.wait()

    w = {}
    pos_ready = set()
    for i, h in enumerate(seq):
        if i + 1 < NCHUNK:
            nh = seq[i + 1]
            prev = i + 1 - NBUF
            if prev >= 0:
                for cp in w[seq[prev]]:
                    cp.wait()
            g[nh] = gathers(nh)
            nq = nh % PERB
            if nq not in pq:
                pq[nq] = pltpu.async_copy(
                    pos_hbm.at[pl.ds(t0 + nq * CH, CH)], pos_v.at[nq], psem
                )
        for cp in g[h]:
            cp.wait()
        q = h % PERB
        if q not in pos_ready:
            pq[q].wait()
            pos_ready.add(q)

        buf = rows_v.at[bufno[h]]

        def j_body(j, _):
            sl = pl.ds(j * LANES, LANES)
            for r in range(CH):
                plsc.addupdate(buf.at[r, sl], pos_v[q, r, sl])
            return _

        lax.fori_loop(0, DV, j_body, 0)

        w[h] = [
            pltpu.async_copy(
                buf, out_hbm.at[pl.ds(row0(h), CH)], wsems[semno[h]]
            )
        ]
    for h in seq[-(NBUF - 1):]:
        for cp in w[h]:
            cp.wait()


@functools.lru_cache(maxsize=None)
def _emb_call():
    return functools.partial(
        pl.kernel,
        out_type=jax.ShapeDtypeStruct((B * T, D), jnp.float32),
        mesh=plsc.VectorSubcoreMesh(
            core_axis_name="c", subcore_axis_name="s", num_cores=NC, num_subcores=NS
        ),
        scratch_types=[
            pltpu.VMEM((B * TW,), jnp.int32),
            pltpu.VMEM((PERB, CH, D), jnp.float32),
            pltpu.VMEM((NBUF, CH, D), jnp.float32),
            pltpu.SemaphoreType.DMA,
            pltpu.SemaphoreType.DMA,
            pltpu.SemaphoreType.DMA,
            pltpu.SemaphoreType.DMA,
            pltpu.SemaphoreType.DMA,
            pltpu.SemaphoreType.DMA,
        ],
    )(_emb_body)


@jax.jit
def kernel(idx, tok_emb, pos_emb):
    b, t = idx.shape
    idx_flat = idx.reshape(b * t).astype(jnp.int32)
    pos2d = pos_emb.reshape(pos_emb.shape[1], pos_emb.shape[2])[:t]
    out = _emb_call()(idx_flat, pos2d, tok_emb)
    return out.reshape(b, t, pos_emb.shape[2])


# speedup vs baseline: 1.0847x; 1.0847x over previous
import functools

import jax
import jax.numpy as jnp
from jax import lax
from jax.experimental import pallas as pl
from jax.experimental.pallas import tpu as pltpu
from jax.experimental.pallas import tpu_sc as plsc

NC = 2
NS = 16
NW = NC * NS

B = 4
T = 2048
D = 768
LANES = 16
DV = D // LANES

TW = T // NW
CW = 8
NCHUNK = TW // CW
NBUF = 3


def _emb_body(
    idx_hbm, pos_hbm, tok_hbm, out_hbm,
    idx_v, pos_v, rows_v,
    isem, gsem0, gsem1, wsem0, wsem1, psem0, psem1,
):
    wid = lax.axis_index("s") * NC + lax.axis_index("c")
    t0 = wid * TW
    gsems = (gsem0, gsem1)
    wsems = (wsem0, wsem1)
    psems = (psem0, psem1)

    idx_cps = [
        pltpu.async_copy(
            idx_hbm.at[pl.ds(b * T + t0, TW)], idx_v.at[pl.ds(b * TW, TW)], isem
        )
        for b in range(B)
    ]
    for cp in idx_cps:
        cp.wait()

    def gathers(h):
        return [
            pltpu.async_copy(
                tok_hbm.at[idx_v.at[pl.ds(b * TW + h * CW, CW)]],
                rows_v.at[h % NBUF].at[pl.ds(b * CW, CW)],
                gsems[h % 2],
            )
            for b in range(B)
        ]

    def pos_copy(h):
        return pltpu.async_copy(
            pos_hbm.at[pl.ds(t0 + h * CW, CW)], pos_v.at[h % 2], psems[h % 2]
        )

    g = {0: gathers(0)}
    p = {0: pos_copy(0)}
    w = {}
    for h in range(NCHUNK):
        if h + 1 < NCHUNK:
            prev = h + 1 - NBUF
            if prev >= 0:
                for cp in w[prev]:
                    cp.wait()
            g[h + 1] = gathers(h + 1)
            p[h + 1] = pos_copy(h + 1)
        for cp in g[h]:
            cp.wait()
        p[h].wait()

        buf = rows_v.at[h % NBUF]
        pb = h % 2

        def j_body(j, _):
            sl = pl.ds(j * LANES, LANES)
            for r in range(CW):
                pv = pos_v[pb, r, sl]
                for b in range(B):
                    plsc.addupdate(buf.at[b * CW + r, sl], pv)
            return _

        lax.fori_loop(0, DV, j_body, 0)

        w[h] = [
            pltpu.async_copy(
                buf.at[pl.ds(b * CW, CW)],
                out_hbm.at[pl.ds(b * T + t0 + h * CW, CW)],
                wsems[h % 2],
            )
            for b in range(B)
        ]
    for h in range(max(0, NCHUNK - NBUF + 1), NCHUNK):
        for cp in w[h]:
            cp.wait()


@functools.lru_cache(maxsize=None)
def _emb_call():
    return functools.partial(
        pl.kernel,
        out_type=jax.ShapeDtypeStruct((B * T, D), jnp.float32),
        mesh=plsc.VectorSubcoreMesh(
            core_axis_name="c", subcore_axis_name="s", num_cores=NC, num_subcores=NS
        ),
        scratch_types=[
            pltpu.VMEM((B * TW,), jnp.int32),
            pltpu.VMEM((2, CW, D), jnp.float32),
            pltpu.VMEM((NBUF, B * CW, D), jnp.float32),
            pltpu.SemaphoreType.DMA,
            pltpu.SemaphoreType.DMA,
            pltpu.SemaphoreType.DMA,
            pltpu.SemaphoreType.DMA,
            pltpu.SemaphoreType.DMA,
            pltpu.SemaphoreType.DMA,
            pltpu.SemaphoreType.DMA,
        ],
    )(_emb_body)


@jax.jit
def kernel(idx, tok_emb, pos_emb):
    b, t = idx.shape
    idx_flat = idx.reshape(b * t).astype(jnp.int32)
    pos2d = pos_emb.reshape(pos_emb.shape[1], pos_emb.shape[2])[:t]
    out = _emb_call()(idx_flat, pos2d, tok_emb)
    return out.reshape(b, t, pos_emb.shape[2])


# speedup vs baseline: 1.0942x; 1.0087x over previous
import functools

import jax
import jax.numpy as jnp
from jax import lax
from jax.experimental import pallas as pl
from jax.experimental.pallas import tpu as pltpu
from jax.experimental.pallas import tpu_sc as plsc

NC = 2
NS = 16
NW = NC * NS

B = 4
T = 2048
D = 768
LANES = 16
DV = D // LANES

TW = T // NW
CW = 8
NCHUNK = TW // CW
NBUF = 3


def _emb_body(
    idx_hbm, pos_hbm, tok_hbm, out_hbm,
    idx_v, pos_v, rows_v,
    isem, gsem0, gsem1, wsem0, wsem1, psem0, psem1,
):
    wid = lax.axis_index("s") * NC + lax.axis_index("c")
    t0 = wid * TW
    gsems = (gsem0, gsem1)
    wsems = (wsem0, wsem1)
    psems = (psem0, psem1)

    i0 = pltpu.async_copy(
        idx_hbm.at[pl.ds(wid * (B * TW), B * CW)], idx_v.at[pl.ds(0, B * CW)], isem
    )
    i1 = pltpu.async_copy(
        idx_hbm.at[pl.ds(wid * (B * TW) + B * CW, B * (TW - CW))],
        idx_v.at[pl.ds(B * CW, B * (TW - CW))],
        isem,
    )
    i0.wait()

    def gathers(h):
        return [
            pltpu.async_copy(
                tok_hbm.at[idx_v.at[pl.ds(h * (B * CW), B * CW)]],
                rows_v.at[h % NBUF],
                gsems[h % 2],
            )
        ]

    def pos_copy(h):
        return pltpu.async_copy(
            pos_hbm.at[pl.ds(t0 + h * CW, CW)], pos_v.at[h % 2], psems[h % 2]
        )

    g = {0: gathers(0)}
    p = {0: pos_copy(0)}
    i1.wait()
    w = {}
    for h in range(NCHUNK):
        if h + 1 < NCHUNK:
            prev = h + 1 - NBUF
            if prev >= 0:
                for cp in w[prev]:
                    cp.wait()
            g[h + 1] = gathers(h + 1)
            p[h + 1] = pos_copy(h + 1)
        for cp in g[h]:
            cp.wait()
        p[h].wait()

        buf = rows_v.at[h % NBUF]
        pb = h % 2

        def j_body(j, _):
            sl = pl.ds(j * LANES, LANES)
            for r in range(CW):
                pv = pos_v[pb, r, sl]
                for b in range(B):
                    plsc.addupdate(buf.at[b * CW + r, sl], pv)
            return _

        lax.fori_loop(0, DV, j_body, 0)

        w[h] = [
            pltpu.async_copy(
                buf.at[pl.ds(b * CW, CW)],
                out_hbm.at[pl.ds(b * T + t0 + h * CW, CW)],
                wsems[h % 2],
            )
            for b in range(B)
        ]
    for h in range(max(0, NCHUNK - NBUF + 1), NCHUNK):
        for cp in w[h]:
            cp.wait()


@functools.lru_cache(maxsize=None)
def _emb_call():
    return functools.partial(
        pl.kernel,
        out_type=jax.ShapeDtypeStruct((B * T, D), jnp.float32),
        mesh=plsc.VectorSubcoreMesh(
            core_axis_name="c", subcore_axis_name="s", num_cores=NC, num_subcores=NS
        ),
        scratch_types=[
            pltpu.VMEM((B * TW,), jnp.int32),
            pltpu.VMEM((2, CW, D), jnp.float32),
            pltpu.VMEM((NBUF, B * CW, D), jnp.float32),
            pltpu.SemaphoreType.DMA,
            pltpu.SemaphoreType.DMA,
            pltpu.SemaphoreType.DMA,
            pltpu.SemaphoreType.DMA,
            pltpu.SemaphoreType.DMA,
            pltpu.SemaphoreType.DMA,
            pltpu.SemaphoreType.DMA,
        ],
    )(_emb_body)


@jax.jit
def kernel(idx, tok_emb, pos_emb):
    b, t = idx.shape
    idx_perm = (
        idx.astype(jnp.int32)
        .reshape(b, NW, NCHUNK, CW)
        .transpose(1, 2, 0, 3)
        .reshape(b * t)
    )
    pos2d = pos_emb.reshape(pos_emb.shape[1], pos_emb.shape[2])[:t]
    out = _emb_call()(idx_perm, pos2d, tok_emb)
    return out.reshape(b, t, pos_emb.shape[2])


# speedup vs baseline: 1.0997x; 1.0050x over previous
import functools

import jax
import jax.numpy as jnp
from jax import lax
from jax.experimental import pallas as pl
from jax.experimental.pallas import tpu as pltpu
from jax.experimental.pallas import tpu_sc as plsc

NC = 2
NS = 16
NW = NC * NS

B = 4
T = 2048
D = 768
LANES = 16
DV = D // LANES

TW = T // NW
CW = 8
NCHUNK = TW // CW
NBUF = 3


def _emb_body(
    idx_hbm, pos_hbm, tok_hbm, out_hbm,
    idx_v, pos_v, rows_v,
    isem, gsem0, gsem1, wsem0, wsem1, psem0, psem1,
):
    wid = lax.axis_index("s") * NC + lax.axis_index("c")
    t0 = wid * TW
    gsems = (gsem0, gsem1)
    wsems = (wsem0, wsem1)
    psems = (psem0, psem1)

    def idx_cp(h, b):
        return pltpu.async_copy(
            idx_hbm.at[pl.ds(b * T + t0 + h * CW, CW)],
            idx_v.at[pl.ds(h * (B * CW) + b * CW, CW)],
            isem,
        )

    i0 = [idx_cp(0, b) for b in range(B)]
    for cp in i0:
        cp.wait()

    def gathers(h):
        return [
            pltpu.async_copy(
                tok_hbm.at[idx_v.at[pl.ds(h * (B * CW), B * CW)]],
                rows_v.at[h % NBUF],
                gsems[h % 2],
            )
        ]

    def pos_copy(h):
        return pltpu.async_copy(
            pos_hbm.at[pl.ds(t0 + h * CW, CW)], pos_v.at[h % 2], psems[h % 2]
        )

    g = {0: gathers(0)}
    p = {0: pos_copy(0)}
    irest = [idx_cp(h, b) for h in range(1, NCHUNK) for b in range(B)]
    for cp in irest:
        cp.wait()
    w = {}
    for h in range(NCHUNK):
        if h + 1 < NCHUNK:
            prev = h + 1 - NBUF
            if prev >= 0:
                for cp in w[prev]:
                    cp.wait()
            g[h + 1] = gathers(h + 1)
            p[h + 1] = pos_copy(h + 1)
        for cp in g[h]:
            cp.wait()
        p[h].wait()

        buf = rows_v.at[h % NBUF]
        pb = h % 2

        def j_body(j, _):
            sl = pl.ds(j * LANES, LANES)
            for r in range(CW):
                pv = pos_v[pb, r, sl]
                for b in range(B):
                    plsc.addupdate(buf.at[b * CW + r, sl], pv)
            return _

        lax.fori_loop(0, DV, j_body, 0)

        w[h] = [
            pltpu.async_copy(
                buf.at[pl.ds(b * CW, CW)],
                out_hbm.at[pl.ds(b * T + t0 + h * CW, CW)],
                wsems[h % 2],
            )
            for b in range(B)
        ]
    for h in range(max(0, NCHUNK - NBUF + 1), NCHUNK):
        for cp in w[h]:
            cp.wait()


@functools.lru_cache(maxsize=None)
def _emb_call():
    return functools.partial(
        pl.kernel,
        out_type=jax.ShapeDtypeStruct((B * T, D), jnp.float32),
        mesh=plsc.VectorSubcoreMesh(
            core_axis_name="c", subcore_axis_name="s", num_cores=NC, num_subcores=NS
        ),
        scratch_types=[
            pltpu.VMEM((B * TW,), jnp.int32),
            pltpu.VMEM((2, CW, D), jnp.float32),
            pltpu.VMEM((NBUF, B * CW, D), jnp.float32),
            pltpu.SemaphoreType.DMA,
            pltpu.SemaphoreType.DMA,
            pltpu.SemaphoreType.DMA,
            pltpu.SemaphoreType.DMA,
            pltpu.SemaphoreType.DMA,
            pltpu.SemaphoreType.DMA,
            pltpu.SemaphoreType.DMA,
        ],
    )(_emb_body)


@jax.jit
def kernel(idx, tok_emb, pos_emb):
    b, t = idx.shape
    idx_flat = idx.astype(jnp.int32).reshape(b * t)
    pos2d = pos_emb.reshape(pos_emb.shape[1], pos_emb.shape[2])[:t]
    out = _emb_call()(idx_flat, pos2d, tok_emb)
    return out.reshape(b, t, pos_emb.shape[2])
